# baseline (speedup 1.0000x reference)
import functools

import jax
import jax.numpy as jnp
from jax import lax
from jax.experimental import pallas as pl
from jax.experimental.pallas import tpu as pltpu
from jax.experimental.pallas import tpu_sc as plsc

S, D, H, F, V = 2048, 1024, 16, 4096, 32000
DH = D // H

NC, NS = 2, 16
NW = NC * NS
ROWS_PER_W = S // NW

QC = 512
NQ = S // QC
FT = 512
NF = F // FT
VT = 1280
NV = V // VT


def _ln(x, g, b):
    m = jnp.mean(x, axis=-1, keepdims=True)
    v = jnp.mean((x - m) ** 2, axis=-1, keepdims=True)
    return (x - m) * lax.rsqrt(v + 1e-5) * g + b


def _bf(x):
    return x.astype(jnp.bfloat16)


_sc_mesh = plsc.VectorSubcoreMesh(core_axis_name="c", subcore_axis_name="s",
                                  num_cores=NC, num_subcores=NS)


@functools.partial(
    pl.kernel,
    out_type=jax.ShapeDtypeStruct((S, D), jnp.float32),
    mesh=_sc_mesh,
    scratch_types=[
        pltpu.VMEM((ROWS_PER_W,), jnp.int32),
        pltpu.VMEM((ROWS_PER_W, D), jnp.float32),
        pltpu.SemaphoreType.DMA,
    ],
)
def _sc_gather(table_hbm, idx_hbm, out_hbm, idx_v, rows_v, sem):
    wid = lax.axis_index("s") * NC + lax.axis_index("c")
    base = wid * ROWS_PER_W
    pltpu.sync_copy(idx_hbm.at[pl.ds(base, ROWS_PER_W)], idx_v)
    pltpu.async_copy(table_hbm.at[idx_v], rows_v, sem).wait()
    pltpu.sync_copy(rows_v, out_hbm.at[pl.ds(base, ROWS_PER_W)])


def _attn_body(x_ref, g_ref, b_ref, wq_ref, wk_ref, wv_ref, ctx_ref,
               hln_s, k_s, v_s):
    h = pl.program_id(0)
    sq = pl.program_id(1)

    @pl.when(jnp.logical_and(h == 0, sq == 0))
    def _():
        hln_s[...] = _bf(_ln(x_ref[...], g_ref[...], b_ref[...]))

    @pl.when(sq == 0)
    def _():
        hln = hln_s[...]
        k_s[...] = _bf(jnp.dot(hln, _bf(wk_ref[0]),
                               preferred_element_type=jnp.float32))
        v_s[...] = _bf(jnp.dot(hln, _bf(wv_ref[0]),
                               preferred_element_type=jnp.float32))

    q = jnp.dot(hln_s[pl.ds(sq * QC, QC), :], _bf(wq_ref[0]),
                preferred_element_type=jnp.float32)
    q = q * (1.0 / (DH ** 0.5))
    s = lax.dot_general(_bf(q), k_s[...], (((1,), (1,)), ((), ())),
                        preferred_element_type=jnp.float32)
    row = sq * QC + lax.broadcasted_iota(jnp.int32, (QC, S), 0)
    col = lax.broadcasted_iota(jnp.int32, (QC, S), 1)
    s = jnp.where(row >= col, s, jnp.float32(-1e9))
    m = jnp.max(s, axis=-1, keepdims=True)
    p = jnp.exp(s - m)
    p = p / jnp.sum(p, axis=-1, keepdims=True)
    ctx_ref[0] = _bf(jnp.dot(_bf(p), v_s[...],
                             preferred_element_type=jnp.float32))


_attn = pl.pallas_call(
    _attn_body,
    grid=(H, NQ),
    in_specs=[
        pl.BlockSpec((S, D), lambda h, sq: (0, 0)),
        pl.BlockSpec((1, D), lambda h, sq: (0, 0)),
        pl.BlockSpec((1, D), lambda h, sq: (0, 0)),
        pl.BlockSpec((1, D, DH), lambda h, sq: (h, 0, 0)),
        pl.BlockSpec((1, D, DH), lambda h, sq: (h, 0, 0)),
        pl.BlockSpec((1, D, DH), lambda h, sq: (h, 0, 0)),
    ],
    out_specs=pl.BlockSpec((1, QC, DH), lambda h, sq: (h, sq, 0)),
    out_shape=jax.ShapeDtypeStruct((H, S, DH), jnp.bfloat16),
    scratch_shapes=[
        pltpu.VMEM((S, D), jnp.bfloat16),
        pltpu.VMEM((S, DH), jnp.bfloat16),
        pltpu.VMEM((S, DH), jnp.bfloat16),
    ],
)


def _proj_body(x_ref, ctx_ref, wo_ref, g_ref, b_ref, x2_ref, h2_ref):
    x2 = x_ref[...]
    for h in range(H):
        x2 += jnp.dot(ctx_ref[h], _bf(wo_ref[h]),
                      preferred_element_type=jnp.float32)
    x2_ref[...] = x2
    h2_ref[...] = _bf(_ln(x2, g_ref[...], b_ref[...]))


_proj = pl.pallas_call(
    _proj_body,
    out_shape=(jax.ShapeDtypeStruct((S, D), jnp.float32),
               jax.ShapeDtypeStruct((S, D), jnp.bfloat16)),
)


def _mlp_body(x2_ref, h2_ref, w1_ref, w2_ref, g_ref, b_ref, hf_ref, acc):
    ft = pl.program_id(0)

    @pl.when(ft == 0)
    def _():
        acc[...] = jnp.zeros_like(acc)

    t = jnp.dot(h2_ref[...], _bf(w1_ref[...]),
                preferred_element_type=jnp.float32)
    t = jax.nn.gelu(t)
    acc[...] += jnp.dot(_bf(t), _bf(w2_ref[...]),
                        preferred_element_type=jnp.float32)

    @pl.when(ft == NF - 1)
    def _():
        hf_ref[...] = _bf(_ln(x2_ref[...] + acc[...], g_ref[...], b_ref[...]))


_mlp = pl.pallas_call(
    _mlp_body,
    grid=(NF,),
    in_specs=[
        pl.BlockSpec((S, D), lambda ft: (0, 0)),
        pl.BlockSpec((S, D), lambda ft: (0, 0)),
        pl.BlockSpec((D, FT), lambda ft: (0, ft)),
        pl.BlockSpec((FT, D), lambda ft: (ft, 0)),
        pl.BlockSpec((1, D), lambda ft: (0, 0)),
        pl.BlockSpec((1, D), lambda ft: (0, 0)),
    ],
    out_specs=pl.BlockSpec((S, D), lambda ft: (0, 0)),
    out_shape=jax.ShapeDtypeStruct((S, D), jnp.bfloat16),
    scratch_shapes=[pltpu.VMEM((S, D), jnp.float32)],
)


def _logits_body(hf_ref, we_ref, out_ref):
    out_ref[...] = lax.dot_general(
        hf_ref[...], _bf(we_ref[...]), (((1,), (1,)), ((), ())),
        preferred_element_type=jnp.float32)


_logits = pl.pallas_call(
    _logits_body,
    grid=(NV,),
    in_specs=[
        pl.BlockSpec((S, D), lambda vt: (0, 0)),
        pl.BlockSpec((VT, D), lambda vt: (vt, 0)),
    ],
    out_specs=pl.BlockSpec((S, VT), lambda vt: (0, vt)),
    out_shape=jax.ShapeDtypeStruct((S, V), jnp.float32),
)


def kernel(input_ids, W_emb, Wq, Wk, Wv, Wo, W1, W2,
           ln1_g, ln1_b, ln2_g, ln2_b, lnf_g, lnf_b):
    ids = input_ids.reshape(S).astype(jnp.int32)
    wq_r = Wq.reshape(D, H, DH).transpose(1, 0, 2)
    wk_r = Wk.reshape(D, H, DH).transpose(1, 0, 2)
    wv_r = Wv.reshape(D, H, DH).transpose(1, 0, 2)
    wo_r = Wo.reshape(H, DH, D)
    x = _sc_gather(W_emb, ids)
    ctx = _attn(x, ln1_g.reshape(1, D), ln1_b.reshape(1, D),
                wq_r, wk_r, wv_r)
    x2, h2 = _proj(x, ctx, wo_r, ln2_g.reshape(1, D), ln2_b.reshape(1, D))
    hf = _mlp(x2, h2, W1, W2, lnf_g.reshape(1, D), lnf_b.reshape(1, D))
    logits = _logits(hf, W_emb)
    return logits.reshape(1, S, V)
